# baseline (speedup 1.0000x reference)
import functools

import jax
import jax.numpy as jnp
from jax import lax
from jax.experimental import pallas as pl
from jax.experimental.pallas import tpu as pltpu
from jax.experimental.pallas import tpu_sc as plsc

_INFO = plsc.get_sparse_core_info()
_NC, _NS = _INFO.num_cores, _INFO.num_subcores
_NW = _NC * _NS


def _make_pack(D, V):
    D2 = 2 * D
    n_full = V // 128
    tail = V - n_full * 128
    per_w = (n_full + _NW - 1) // _NW

    mesh = plsc.VectorSubcoreMesh(core_axis_name="c", subcore_axis_name="s")

    @functools.partial(
        pl.kernel,
        out_type=jax.ShapeDtypeStruct((V // 2, D2), jnp.float32),
        mesh=mesh,
        scratch_types=[
            pltpu.VMEM((D, 128), jnp.float32),
            pltpu.VMEM((64, D2), jnp.float32),
            pltpu.VMEM((64, D), jnp.float32),
        ],
        compiler_params=pltpu.CompilerParams(needs_layout_passes=False),
    )
    def pack_kernel(wt_hbm, wtail_hbm, w2_hbm, inbuf, outbuf, tb):
        wid = lax.axis_index("s") * _NC + lax.axis_index("c")
        iota = lax.iota(jnp.int32, 16)

        def blk_body(i, carry):
            j = wid + i * _NW

            @pl.when(j < n_full)
            def _():
                v0 = pl.multiple_of(j * 128, 128)
                pltpu.sync_copy(wt_hbm.at[pl.ds(0, D), pl.ds(v0, 128)], inbuf)

                def r_body(r, carry2):
                    for h in range(2):
                        col = 2 * r + h
                        for g in range(D // 16):
                            rows = iota + (g * 16)
                            cols = jnp.full((16,), col, jnp.int32)
                            outbuf[r, pl.ds(h * D + g * 16, 16)] = (
                                plsc.load_gather(inbuf, [rows, cols])
                            )
                    return carry2

                lax.fori_loop(0, 64, r_body, 0)
                o0 = pl.multiple_of(j * 64, 64)
                pltpu.sync_copy(outbuf, w2_hbm.at[pl.ds(o0, 64)])

            return carry

        lax.fori_loop(0, per_w, blk_body, 0)

        if tail:
            @pl.when(wid == 0)
            def _():
                pltpu.sync_copy(wtail_hbm, tb)

                def r_body(r, carry2):
                    for h in range(2):
                        for g in range(D // 16):
                            outbuf[r, pl.ds(h * D + g * 16, 16)] = (
                                tb[2 * r + h, pl.ds(g * 16, 16)]
                            )
                    return carry2

                lax.fori_loop(0, tail // 2, r_body, 0)
                o0 = pl.multiple_of(n_full * 64, 8)
                pltpu.sync_copy(
                    outbuf.at[pl.ds(0, tail // 2)],
                    w2_hbm.at[pl.ds(o0, tail // 2)],
                )

    return pack_kernel


def _make_gather(V2, D, B, L):
    CB = B // _NW
    LBLK = 8
    n_lb = L // LBLK
    D2 = 2 * D

    mesh = plsc.VectorSubcoreMesh(core_axis_name="c", subcore_axis_name="s")

    @functools.partial(
        pl.kernel,
        out_type=jax.ShapeDtypeStruct((L, D, B), jnp.float32),
        mesh=mesh,
        scratch_types=[
            pltpu.VMEM((LBLK, CB), jnp.int32),
            pltpu.VMEM((LBLK, CB), jnp.int32),
            pltpu.VMEM((CB,), jnp.int32),
            pltpu.VMEM((CB, D2), jnp.float32),
            pltpu.VMEM((D, CB), jnp.float32),
            pltpu.SemaphoreType.DMA,
        ],
        compiler_params=pltpu.CompilerParams(needs_layout_passes=False),
    )
    def gather_kernel(w_hbm, idx_hbm, out_hbm, idxr, idxg, pcol, pair, slab, gsem):
        wid = lax.axis_index("s") * _NC + lax.axis_index("c")
        b0 = pl.multiple_of(wid * CB, CB)
        iota = lax.iota(jnp.int32, 16)

        def lb_body(lb, carry):
            l0 = pl.multiple_of(lb * LBLK, LBLK)
            pltpu.sync_copy(idx_hbm.at[pl.ds(l0, LBLK), pl.ds(b0, CB)], idxr)
            for r in range(LBLK):
                for g in range(CB // 16):
                    idxg[r, pl.ds(g * 16, 16)] = (
                        idxr[r, pl.ds(g * 16, 16)] >> 1
                    )

            def dl_body(dl, carry2):
                l = l0 + dl
                pltpu.async_copy(w_hbm.at[idxg.at[dl]], pair, gsem).wait()
                for g in range(CB // 16):
                    qv = idxr[dl, pl.ds(g * 16, 16)]
                    pcol[pl.ds(g * 16, 16)] = (qv & 1) << 6

                def d_body(dd, carry3):
                    for g in range(CB // 16):
                        rvec = iota + (g * 16)
                        col = pcol[pl.ds(g * 16, 16)] + dd
                        slab[dd, pl.ds(g * 16, 16)] = plsc.load_gather(
                            pair, [rvec, col]
                        )
                    return carry3

                lax.fori_loop(0, D, d_body, 0)
                pltpu.sync_copy(
                    slab, out_hbm.at[l, pl.ds(0, D), pl.ds(b0, CB)]
                )
                return carry2

            lax.fori_loop(0, LBLK, dl_body, 0)
            return carry

        lax.fori_loop(0, n_lb, lb_body, 0)

    return gather_kernel


def kernel(q, W):
    B, L = q.shape
    V, D = W.shape
    tail = V - (V // 128) * 128
    W2 = _make_pack(D, V)(W.T, W[V - tail:, :])
    qT = q.T.astype(jnp.int32)
    out = _make_gather(V // 2, D, B, L)(W2, qT)
    return jnp.transpose(out, (2, 0, 1))


# speedup vs baseline: 4.3967x; 4.3967x over previous
import functools

import jax
import jax.numpy as jnp
from jax import lax
from jax.experimental import pallas as pl
from jax.experimental.pallas import tpu as pltpu
from jax.experimental.pallas import tpu_sc as plsc

_GCHUNK = 512
_IDXBLK = 2 * _GCHUNK


def _make_gather(V, D, N):
    info = plsc.get_sparse_core_info()
    NC, NS = info.num_cores, info.num_subcores
    NW = NC * NS
    assert N % (NW * _IDXBLK) == 0
    n_per_w = N // NW
    n_it = n_per_w // _IDXBLK

    mesh = plsc.VectorSubcoreMesh(core_axis_name="c", subcore_axis_name="s")

    @functools.partial(
        pl.kernel,
        out_type=jax.ShapeDtypeStruct((N, 2 * D), jnp.float32),
        mesh=mesh,
        scratch_types=[
            pltpu.VMEM((_IDXBLK,), jnp.int32),
            pltpu.VMEM((_GCHUNK, D), jnp.float32),
            pltpu.VMEM((_GCHUNK, D), jnp.float32),
            pltpu.SemaphoreType.DMA,
            pltpu.SemaphoreType.DMA,
        ],
        compiler_params=pltpu.CompilerParams(use_tc_tiling_on_sc=False),
    )
    def gather_kernel(w_hbm, idx_hbm, out_hbm, idx_buf, rows0, rows1, gsem, wsem):
        wid = lax.axis_index("s") * NC + lax.axis_index("c")
        wbase = wid * n_per_w
        bufs = (rows0, rows1)

        def body(i, carry):
            base = pl.multiple_of(wbase + i * _IDXBLK, _IDXBLK)
            pltpu.sync_copy(idx_hbm.at[pl.ds(base, _IDXBLK)], idx_buf)
            for s in range(2):
                buf = bufs[s]
                @pl.when(i > 0)
                def _():
                    pltpu.make_async_copy(
                        buf, out_hbm.at[pl.ds(0, _GCHUNK), pl.ds(0, D)], wsem
                    ).wait()
                pltpu.async_copy(
                    w_hbm.at[idx_buf.at[pl.ds(s * _GCHUNK, _GCHUNK)]],
                    buf,
                    gsem,
                ).wait()
                pltpu.async_copy(
                    buf,
                    out_hbm.at[pl.ds(base + s * _GCHUNK, _GCHUNK), pl.ds(0, D)],
                    wsem,
                )
            return carry

        lax.fori_loop(0, n_it, body, 0)
        for s in range(2):
            pltpu.make_async_copy(
                bufs[s], out_hbm.at[pl.ds(0, _GCHUNK), pl.ds(0, D)], wsem
            ).wait()

    return gather_kernel


def kernel(q, W):
    B, L = q.shape
    V, D = W.shape
    N = B * L
    idx = q.reshape(N).astype(jnp.int32)
    out = _make_gather(V, D, N)(W, idx)
    return out[:, :D].reshape(B, L, D)
